# Initial kernel scaffold; baseline (speedup 1.0000x reference)
import jax
import jax.numpy as jnp
from jax.experimental import pallas as pl


def kernel(x1, x2, edge_index1, edge_index2, graph_ids1, graph_ids2, W_gc0, b_gc0, W_res0, b_res0, g_bn0, beta_bn0, W_gc1, b_gc1, W_res1, b_res1, g_bn1, beta_bn1, W_aw, b_aw, W_p1, b_p1, g_bnp, beta_bnp, W_p2, b_p2, W_fc, b_fc, g_bnf, beta_bnf, W_out, b_out):
    raise NotImplementedError("write your pallas kernel here")


# speedup vs baseline: 1.1473x; 1.1473x over previous
import functools

import jax
import jax.numpy as jnp
from jax import lax
from jax.experimental import pallas as pl
from jax.experimental.pallas import tpu as pltpu

N = 50000
E = 1600000
B = 1024
D_IN = 32
H = 128

ROWS = 1000


def _t1_body(nparts, i_ref_args):
    pass


def _t1_kernel(*refs, nparts):
    agg_refs = refs[:nparts]
    h_ref, wg_ref, bg_ref, wr_ref, br_ref, z_ref, st_ref = refs[nparts:]
    i = pl.program_id(0)
    agg = agg_refs[0][...]
    for r in agg_refs[1:]:
        agg = agg + r[...]
    a = jnp.dot(agg, wg_ref[...], preferred_element_type=jnp.float32)
    a = jnp.maximum(a + bg_ref[...], 0.0)
    r = jnp.dot(h_ref[...], wr_ref[...], preferred_element_type=jnp.float32)
    r = jnp.maximum(r + br_ref[...], 0.0)
    z = a + r
    z_ref[...] = z

    @pl.when(i == 0)
    def _():
        st_ref[...] = jnp.zeros_like(st_ref)

    st_ref[0:1, :] += jnp.sum(z, axis=0, keepdims=True)
    st_ref[1:2, :] += jnp.sum(z * z, axis=0, keepdims=True)


def _t1(agg_parts, h, wg, bg, wr, br):
    nparts = len(agg_parts)
    din = h.shape[1]
    grid = (N // ROWS,)
    in_specs = (
        [pl.BlockSpec((ROWS, din), lambda i: (i, 0))] * nparts
        + [
            pl.BlockSpec((ROWS, din), lambda i: (i, 0)),
            pl.BlockSpec((din, H), lambda i: (0, 0)),
            pl.BlockSpec((1, H), lambda i: (0, 0)),
            pl.BlockSpec((din, H), lambda i: (0, 0)),
            pl.BlockSpec((1, H), lambda i: (0, 0)),
        ]
    )
    out_specs = [
        pl.BlockSpec((ROWS, H), lambda i: (i, 0)),
        pl.BlockSpec((8, H), lambda i: (0, 0)),
    ]
    z, st = pl.pallas_call(
        functools.partial(_t1_kernel, nparts=nparts),
        grid=grid,
        in_specs=in_specs,
        out_specs=out_specs,
        out_shape=[
            jax.ShapeDtypeStruct((N, H), jnp.float32),
            jax.ShapeDtypeStruct((8, H), jnp.float32),
        ],
    )(*agg_parts, h, wg, bg.reshape(1, H), wr, br.reshape(1, H))
    return z, st


def _t2a_kernel(z_ref, st_ref, g_ref, beta_ref, hg_ref):
    s = st_ref[0:1, :]
    ss = st_ref[1:2, :]
    m = s / N
    v = ss / N - m * m
    scale = g_ref[...] * lax.rsqrt(v + 1e-5)
    shift = beta_ref[...] - m * scale
    hcur = z_ref[...] * scale + shift
    for k in range(4):
        hg_ref[k] = hcur[:, 32 * k:32 * (k + 1)]


def _t2a(z, st, g, beta):
    return pl.pallas_call(
        _t2a_kernel,
        grid=(N // ROWS,),
        in_specs=[
            pl.BlockSpec((ROWS, H), lambda i: (i, 0)),
            pl.BlockSpec((8, H), lambda i: (0, 0)),
            pl.BlockSpec((1, H), lambda i: (0, 0)),
            pl.BlockSpec((1, H), lambda i: (0, 0)),
        ],
        out_specs=pl.BlockSpec((4, ROWS, 32), lambda i: (0, i, 0)),
        out_shape=jax.ShapeDtypeStruct((4, N, 32), jnp.float32),
    )(z, st, g.reshape(1, H), beta.reshape(1, H))


def _t2b_kernel(z_ref, st_ref, g_ref, beta_ref, waw_ref, baw_ref,
                h2_ref, y_ref):
    s = st_ref[0:1, :]
    ss = st_ref[1:2, :]
    m = s / N
    v = ss / N - m * m
    scale = g_ref[...] * lax.rsqrt(v + 1e-5)
    shift = beta_ref[...] - m * scale
    h2 = z_ref[...] * scale + shift
    h2_ref[...] = h2
    wv = jnp.dot(h2, waw_ref[...], preferred_element_type=jnp.float32)
    wv = jax.nn.sigmoid(wv + baw_ref[0, 0])
    y_ref[...] = wv * h2


def _t2b(z, st, g, beta, waw, baw):
    return pl.pallas_call(
        _t2b_kernel,
        grid=(N // ROWS,),
        in_specs=[
            pl.BlockSpec((ROWS, H), lambda i: (i, 0)),
            pl.BlockSpec((8, H), lambda i: (0, 0)),
            pl.BlockSpec((1, H), lambda i: (0, 0)),
            pl.BlockSpec((1, H), lambda i: (0, 0)),
            pl.BlockSpec((H, 1), lambda i: (0, 0)),
            pl.BlockSpec((1, 1), lambda i: (0, 0), memory_space=pltpu.SMEM),
        ],
        out_specs=[
            pl.BlockSpec((ROWS, H), lambda i: (i, 0)),
            pl.BlockSpec((ROWS, H), lambda i: (i, 0)),
        ],
        out_shape=[
            jax.ShapeDtypeStruct((N, H), jnp.float32),
            jax.ShapeDtypeStruct((N, H), jnp.float32),
        ],
    )(z, st, g.reshape(1, H), beta.reshape(1, H), waw, baw.reshape(1, 1))


def _t3_kernel(ws_ref, mx_ref, wp1_ref, bp1_ref, g_ref, beta_ref,
               wp2_ref, bp2_ref, s_ref):
    ws = ws_ref[0]
    mx = mx_ref[0]
    mx = jnp.where(jnp.isfinite(mx), mx, 0.0)
    gf = jnp.concatenate([ws, mx], axis=1)
    q = jnp.dot(gf, wp1_ref[...], preferred_element_type=jnp.float32)
    q = jnp.maximum(q + bp1_ref[...], 0.0)
    m = jnp.mean(q, axis=0, keepdims=True)
    v = jnp.mean(q * q, axis=0, keepdims=True) - m * m
    q = g_ref[...] * (q - m) * lax.rsqrt(v + 1e-5) + beta_ref[...]
    s_ref[0] = jnp.dot(q, wp2_ref[...],
                       preferred_element_type=jnp.float32) + bp2_ref[...]


def _t3(ws, mx, wp1, bp1, g, beta, wp2, bp2):
    h2c = 2 * H
    return pl.pallas_call(
        _t3_kernel,
        grid=(2,),
        in_specs=[
            pl.BlockSpec((1, B, H), lambda i: (i, 0, 0)),
            pl.BlockSpec((1, B, H), lambda i: (i, 0, 0)),
            pl.BlockSpec((h2c, h2c), lambda i: (0, 0)),
            pl.BlockSpec((1, h2c), lambda i: (0, 0)),
            pl.BlockSpec((1, h2c), lambda i: (0, 0)),
            pl.BlockSpec((1, h2c), lambda i: (0, 0)),
            pl.BlockSpec((h2c, h2c), lambda i: (0, 0)),
            pl.BlockSpec((1, h2c), lambda i: (0, 0)),
        ],
        out_specs=pl.BlockSpec((1, B, h2c), lambda i: (i, 0, 0)),
        out_shape=jax.ShapeDtypeStruct((2, B, h2c), jnp.float32),
    )(ws, mx, wp1, bp1.reshape(1, h2c), g.reshape(1, h2c),
      beta.reshape(1, h2c), wp2, bp2.reshape(1, h2c))


def _t4_kernel(s_ref, wfc_ref, bfc_ref, g_ref, beta_ref, wout_ref, bout_ref,
               o_ref):
    gf = jnp.concatenate([s_ref[0], s_ref[1]], axis=1)
    z = jnp.dot(gf, wfc_ref[...], preferred_element_type=jnp.float32)
    z = z + bfc_ref[...]
    m = jnp.mean(z, axis=0, keepdims=True)
    v = jnp.mean(z * z, axis=0, keepdims=True) - m * m
    z = g_ref[...] * (z - m) * lax.rsqrt(v + 1e-5) + beta_ref[...]
    z = jnp.maximum(z, 0.0)
    o = jnp.dot(z, wout_ref[...], preferred_element_type=jnp.float32)
    o_ref[...] = jax.nn.sigmoid(o + bout_ref[0, 0])


def _t4(s, wfc, bfc, g, beta, wout, bout):
    return pl.pallas_call(
        _t4_kernel,
        in_specs=[
            pl.BlockSpec((2, B, 2 * H), lambda: (0, 0, 0)),
            pl.BlockSpec((4 * H, 8 * H), lambda: (0, 0)),
            pl.BlockSpec((1, 8 * H), lambda: (0, 0)),
            pl.BlockSpec((1, 8 * H), lambda: (0, 0)),
            pl.BlockSpec((1, 8 * H), lambda: (0, 0)),
            pl.BlockSpec((8 * H, 1), lambda: (0, 0)),
            pl.BlockSpec((1, 1), lambda: (0, 0), memory_space=pltpu.SMEM),
        ],
        out_specs=pl.BlockSpec((B, 1), lambda: (0, 0)),
        out_shape=jax.ShapeDtypeStruct((B, 1), jnp.float32),
    )(s, wfc, bfc.reshape(1, 8 * H), g.reshape(1, 8 * H),
      beta.reshape(1, 8 * H), wout, bout.reshape(1, 1))


def _branch(x, ei, gid, p):
    src = ei[0]
    dst = ei[1]
    agg0 = jax.ops.segment_sum(x[src], dst, num_segments=N)
    z0, st0 = _t1([agg0], x, p['W_gc0'], p['b_gc0'], p['W_res0'], p['b_res0'])
    h1g = _t2a(z0, st0, p['g_bn0'], p['beta_bn0'])
    h1 = jnp.concatenate([h1g[k] for k in range(4)], axis=1)
    agg1 = jax.ops.segment_sum(h1[src], dst, num_segments=N)
    z1, st1 = _t1([agg1], h1, p['W_gc1'], p['b_gc1'], p['W_res1'], p['b_res1'])
    h2, y = _t2b(z1, st1, p['g_bn1'], p['beta_bn1'], p['W_aw'], p['b_aw'])
    ws = jax.ops.segment_sum(y, gid, num_segments=B)
    mx = jax.ops.segment_max(h2, gid, num_segments=B)
    return ws, mx


def kernel(x1, x2, edge_index1, edge_index2, graph_ids1, graph_ids2,
           W_gc0, b_gc0, W_res0, b_res0, g_bn0, beta_bn0,
           W_gc1, b_gc1, W_res1, b_res1, g_bn1, beta_bn1,
           W_aw, b_aw, W_p1, b_p1, g_bnp, beta_bnp, W_p2, b_p2,
           W_fc, b_fc, g_bnf, beta_bnf, W_out, b_out):
    p = dict(W_gc0=W_gc0, b_gc0=b_gc0, W_res0=W_res0, b_res0=b_res0,
             g_bn0=g_bn0, beta_bn0=beta_bn0,
             W_gc1=W_gc1, b_gc1=b_gc1, W_res1=W_res1, b_res1=b_res1,
             g_bn1=g_bn1, beta_bn1=beta_bn1, W_aw=W_aw, b_aw=b_aw)
    ws1, mx1 = _branch(x1, edge_index1, graph_ids1, p)
    ws2, mx2 = _branch(x2, edge_index2, graph_ids2, p)
    ws = jnp.stack([ws1, ws2])
    mx = jnp.stack([mx1, mx2])
    s = _t3(ws, mx, W_p1, b_p1, g_bnp, beta_bnp, W_p2, b_p2)
    o = _t4(s, W_fc, b_fc, g_bnf, beta_bnf, W_out, b_out)
    return o[:, 0]


# speedup vs baseline: 3.6108x; 3.1472x over previous
import functools

import jax
import jax.numpy as jnp
from jax import lax
from jax.experimental import pallas as pl
from jax.experimental.pallas import tpu as pltpu
from jax.experimental.pallas import tpu_sc as plsc

N = 50000
E = 1600000
B = 1024
D_IN = 32
H = 128

ROWS = 1000
NB2 = 12800
E2 = NB2 * 128
NPAD = N + 128
ACC_ROWS = 50048


def _sc_agg(table, srcp, dstp, zin):
    mesh = plsc.VectorSubcoreMesh(core_axis_name="c", subcore_axis_name="s")

    @functools.partial(
        pl.kernel,
        out_type=jax.ShapeDtypeStruct((2, N, 32), jnp.float32),
        mesh=mesh,
        compiler_params=pltpu.CompilerParams(use_tc_tiling_on_sc=False),
        scratch_types=[
            pltpu.VMEM_SHARED((ACC_ROWS, 32), jnp.float32),
            pltpu.VMEM((16, 128), jnp.int32),
            pltpu.VMEM((16, 128), jnp.int32),
            pltpu.VMEM((4, 128, 32), jnp.float32),
            pltpu.SemaphoreType.DMA((4,)),
        ],
    )
    def k(table_hbm, src_hbm, dst_hbm, zin_hbm, out_hbm,
          acc, idx_s, idx_d, rows, gsem):
        c = lax.axis_index("c")
        s = lax.axis_index("s")
        zoff = pl.multiple_of(s * 3128, 8)
        pltpu.sync_copy(zin_hbm, acc.at[pl.ds(zoff, 3128)])
        plsc.subcore_barrier()

        base = (c * 16 + s) * 400

        def group(g, carry):
            g0 = pl.multiple_of(base + g * 16, 8)
            pltpu.sync_copy(src_hbm.at[pl.ds(g0, 16)], idx_s)
            pltpu.sync_copy(dst_hbm.at[pl.ds(g0, 16)], idx_d)
            hs = {}
            for j in range(4):
                hs[j] = pltpu.async_copy(
                    table_hbm.at[idx_s.at[j]], rows.at[j], gsem.at[j])
            for j in range(16):
                hs[j].wait()
                pltpu.sync_copy(rows.at[j % 4], acc.at[idx_d.at[j]],
                                add=True)
                if j + 4 < 16:
                    jn = j + 4
                    hs[jn] = pltpu.async_copy(
                        table_hbm.at[idx_s.at[jn]], rows.at[jn % 4],
                        gsem.at[jn % 4])
            return carry

        lax.fori_loop(0, 25, group, 0)
        plsc.subcore_barrier()
        ooff = pl.multiple_of(s * 3128, 8)

        @pl.when(s < 15)
        def _():
            pltpu.sync_copy(acc.at[pl.ds(ooff, 3128)],
                            out_hbm.at[c, pl.ds(ooff, 3128)])

        @pl.when(s == 15)
        def _():
            pltpu.sync_copy(acc.at[pl.ds(15 * 3128, N - 15 * 3128)],
                            out_hbm.at[c, pl.ds(15 * 3128, N - 15 * 3128)])

    return k(table, srcp, dstp, zin)


def _extract_i32(buf, i):
    win = buf[pl.ds(i, 16)]
    return win[0]


def _sc_readout(gid, y, h2):
    mesh = plsc.VectorSubcoreMesh(core_axis_name="c", subcore_axis_name="s")

    @functools.partial(
        pl.kernel,
        out_type=[
            jax.ShapeDtypeStruct((B, H), jnp.float32),
            jax.ShapeDtypeStruct((B, H), jnp.float32),
        ],
        mesh=mesh,
        scratch_types=[
            pltpu.VMEM((N + 16,), jnp.int32),
            pltpu.VMEM((128, H), jnp.float32),
            pltpu.VMEM((128, H), jnp.float32),
            pltpu.VMEM((32, H), jnp.float32),
            pltpu.VMEM((32, H), jnp.float32),
        ],
    )
    def k(gid_hbm, y_hbm, h2_hbm, ws_hbm, mx_hbm,
          gid_buf, ybuf, hbuf, ws_loc, mx_loc):
        c = lax.axis_index("c")
        s = lax.axis_index("s")
        w = c * 16 + s
        pltpu.sync_copy(gid_hbm, gid_buf.at[pl.ds(0, N)])

        def ini(r, carry):
            for cp in range(8):
                sl = pl.ds(cp * 16, 16)
                ws_loc[r, sl] = jnp.zeros((16,), jnp.float32)
                mx_loc[r, sl] = jnp.full((16,), -jnp.inf, jnp.float32)
            return carry

        lax.fori_loop(0, 32, ini, 0)

        def search(target):
            def it(t, lohi):
                lo, hi = lohi
                mid = (lo + hi) // 2
                v = _extract_i32(gid_buf, mid)
                less = v < target
                return (jnp.where(less, mid + 1, lo),
                        jnp.where(less, hi, mid))
            lo, _ = lax.fori_loop(0, 17, it, (0, N))
            return lo

        g_lo = w * 32
        r0 = search(g_lo)
        r1 = search(g_lo + 32)
        c0 = (r0 // 8) * 8
        nch = (r1 - c0 + 127) // 128

        def chunk(ci, carry):
            cstart = pl.multiple_of(c0 + ci * 128, 8)
            pltpu.sync_copy(y_hbm.at[pl.ds(cstart, 128)], ybuf)
            pltpu.sync_copy(h2_hbm.at[pl.ds(cstart, 128)], hbuf)
            rl_lo = jnp.maximum(0, r0 - cstart)
            rl_hi = jnp.minimum(128, r1 - cstart)

            def row(rl, carry2):
                g = _extract_i32(gid_buf, cstart + rl) - g_lo
                for cp in range(8):
                    sl = pl.ds(cp * 16, 16)
                    ws_loc[g, sl] = ws_loc[g, sl] + ybuf[rl, sl]
                    mx_loc[g, sl] = jnp.maximum(mx_loc[g, sl], hbuf[rl, sl])
                return carry2

            lax.fori_loop(rl_lo, rl_hi, row, 0)
            return carry

        lax.fori_loop(0, nch, chunk, 0)
        pltpu.sync_copy(ws_loc, ws_hbm.at[pl.ds(g_lo, 32)])
        pltpu.sync_copy(mx_loc, mx_hbm.at[pl.ds(g_lo, 32)])

    return k(gid, y, h2)


def _t1_kernel(*refs, ngroups, nh):
    agg_refs = refs[:ngroups]
    h_refs = refs[ngroups:ngroups + nh]
    wg_ref, bg_ref, wr_ref, br_ref, z_ref, st_ref = refs[ngroups + nh:]
    i = pl.program_id(0)
    agg = jnp.concatenate([r[0] + r[1] for r in agg_refs], axis=1)
    h = jnp.concatenate([r[...] for r in h_refs], axis=1)
    a = jnp.dot(agg, wg_ref[...], preferred_element_type=jnp.float32)
    a = jnp.maximum(a + bg_ref[...], 0.0)
    r = jnp.dot(h, wr_ref[...], preferred_element_type=jnp.float32)
    r = jnp.maximum(r + br_ref[...], 0.0)
    z = a + r
    z_ref[...] = z

    @pl.when(i == 0)
    def _():
        st_ref[...] = jnp.zeros_like(st_ref)

    st_ref[0:1, :] += jnp.sum(z, axis=0, keepdims=True)
    st_ref[1:2, :] += jnp.sum(z * z, axis=0, keepdims=True)


def _t1(agg_parts, hs, wg, bg, wr, br):
    ngroups = len(agg_parts)
    nh = len(hs)
    din = 32 * nh
    grid = (N // ROWS,)
    in_specs = (
        [pl.BlockSpec((2, ROWS, 32), lambda i: (0, i, 0))] * ngroups
        + [pl.BlockSpec((ROWS, 32), lambda i: (i, 0))] * nh
        + [
            pl.BlockSpec((din, H), lambda i: (0, 0)),
            pl.BlockSpec((1, H), lambda i: (0, 0)),
            pl.BlockSpec((din, H), lambda i: (0, 0)),
            pl.BlockSpec((1, H), lambda i: (0, 0)),
        ]
    )
    out_specs = [
        pl.BlockSpec((ROWS, H), lambda i: (i, 0)),
        pl.BlockSpec((8, H), lambda i: (0, 0)),
    ]
    z, st = pl.pallas_call(
        functools.partial(_t1_kernel, ngroups=ngroups, nh=nh),
        grid=grid,
        in_specs=in_specs,
        out_specs=out_specs,
        out_shape=[
            jax.ShapeDtypeStruct((N, H), jnp.float32),
            jax.ShapeDtypeStruct((8, H), jnp.float32),
        ],
    )(*agg_parts, *hs, wg, bg.reshape(1, H), wr, br.reshape(1, H))
    return z, st


def _t2a_kernel(z_ref, st_ref, g_ref, beta_ref, *hg_refs):
    s = st_ref[0:1, :]
    ss = st_ref[1:2, :]
    m = s / N
    v = ss / N - m * m
    scale = g_ref[...] * lax.rsqrt(v + 1e-5)
    shift = beta_ref[...] - m * scale
    hcur = z_ref[...] * scale + shift
    for k in range(4):
        hg_refs[k][...] = hcur[:, 32 * k:32 * (k + 1)]


def _t2a(z, st, g, beta):
    return pl.pallas_call(
        _t2a_kernel,
        grid=(N // ROWS,),
        in_specs=[
            pl.BlockSpec((ROWS, H), lambda i: (i, 0)),
            pl.BlockSpec((8, H), lambda i: (0, 0)),
            pl.BlockSpec((1, H), lambda i: (0, 0)),
            pl.BlockSpec((1, H), lambda i: (0, 0)),
        ],
        out_specs=[pl.BlockSpec((ROWS, 32), lambda i: (i, 0))] * 4,
        out_shape=[jax.ShapeDtypeStruct((N, 32), jnp.float32)] * 4,
    )(z, st, g.reshape(1, H), beta.reshape(1, H))


def _t2b_kernel(z_ref, st_ref, g_ref, beta_ref, waw_ref, baw_ref,
                h2_ref, y_ref):
    s = st_ref[0:1, :]
    ss = st_ref[1:2, :]
    m = s / N
    v = ss / N - m * m
    scale = g_ref[...] * lax.rsqrt(v + 1e-5)
    shift = beta_ref[...] - m * scale
    h2 = z_ref[...] * scale + shift
    h2_ref[...] = h2
    wv = jnp.dot(h2, waw_ref[...], preferred_element_type=jnp.float32)
    wv = jax.nn.sigmoid(wv + baw_ref[0, 0])
    y_ref[...] = wv * h2


def _t2b(z, st, g, beta, waw, baw):
    return pl.pallas_call(
        _t2b_kernel,
        grid=(N // ROWS,),
        in_specs=[
            pl.BlockSpec((ROWS, H), lambda i: (i, 0)),
            pl.BlockSpec((8, H), lambda i: (0, 0)),
            pl.BlockSpec((1, H), lambda i: (0, 0)),
            pl.BlockSpec((1, H), lambda i: (0, 0)),
            pl.BlockSpec((H, 1), lambda i: (0, 0)),
            pl.BlockSpec((1, 1), lambda i: (0, 0), memory_space=pltpu.SMEM),
        ],
        out_specs=[
            pl.BlockSpec((ROWS, H), lambda i: (i, 0)),
            pl.BlockSpec((ROWS, H), lambda i: (i, 0)),
        ],
        out_shape=[
            jax.ShapeDtypeStruct((NPAD, H), jnp.float32),
            jax.ShapeDtypeStruct((NPAD, H), jnp.float32),
        ],
    )(z, st, g.reshape(1, H), beta.reshape(1, H), waw, baw.reshape(1, 1))


def _t3_kernel(ws_ref, mx_ref, wp1_ref, bp1_ref, g_ref, beta_ref,
               wp2_ref, bp2_ref, s_ref):
    ws = ws_ref[0]
    mx = mx_ref[0]
    mx = jnp.where(jnp.isfinite(mx), mx, 0.0)
    gf = jnp.concatenate([ws, mx], axis=1)
    q = jnp.dot(gf, wp1_ref[...], preferred_element_type=jnp.float32)
    q = jnp.maximum(q + bp1_ref[...], 0.0)
    m = jnp.mean(q, axis=0, keepdims=True)
    v = jnp.mean(q * q, axis=0, keepdims=True) - m * m
    q = g_ref[...] * (q - m) * lax.rsqrt(v + 1e-5) + beta_ref[...]
    s_ref[0] = jnp.dot(q, wp2_ref[...],
                       preferred_element_type=jnp.float32) + bp2_ref[...]


def _t3(ws, mx, wp1, bp1, g, beta, wp2, bp2):
    h2c = 2 * H
    return pl.pallas_call(
        _t3_kernel,
        grid=(2,),
        in_specs=[
            pl.BlockSpec((1, B, H), lambda i: (i, 0, 0)),
            pl.BlockSpec((1, B, H), lambda i: (i, 0, 0)),
            pl.BlockSpec((h2c, h2c), lambda i: (0, 0)),
            pl.BlockSpec((1, h2c), lambda i: (0, 0)),
            pl.BlockSpec((1, h2c), lambda i: (0, 0)),
            pl.BlockSpec((1, h2c), lambda i: (0, 0)),
            pl.BlockSpec((h2c, h2c), lambda i: (0, 0)),
            pl.BlockSpec((1, h2c), lambda i: (0, 0)),
        ],
        out_specs=pl.BlockSpec((1, B, h2c), lambda i: (i, 0, 0)),
        out_shape=jax.ShapeDtypeStruct((2, B, h2c), jnp.float32),
    )(ws, mx, wp1, bp1.reshape(1, h2c), g.reshape(1, h2c),
      beta.reshape(1, h2c), wp2, bp2.reshape(1, h2c))


def _t4_kernel(s_ref, wfc_ref, bfc_ref, g_ref, beta_ref, wout_ref, bout_ref,
               o_ref):
    gf = jnp.concatenate([s_ref[0], s_ref[1]], axis=1)
    z = jnp.dot(gf, wfc_ref[...], preferred_element_type=jnp.float32)
    z = z + bfc_ref[...]
    m = jnp.mean(z, axis=0, keepdims=True)
    v = jnp.mean(z * z, axis=0, keepdims=True) - m * m
    z = g_ref[...] * (z - m) * lax.rsqrt(v + 1e-5) + beta_ref[...]
    z = jnp.maximum(z, 0.0)
    o = jnp.dot(z, wout_ref[...], preferred_element_type=jnp.float32)
    o_ref[...] = jax.nn.sigmoid(o + bout_ref[0, 0])


def _t4(s, wfc, bfc, g, beta, wout, bout):
    return pl.pallas_call(
        _t4_kernel,
        in_specs=[
            pl.BlockSpec((2, B, 2 * H), lambda: (0, 0, 0)),
            pl.BlockSpec((4 * H, 8 * H), lambda: (0, 0)),
            pl.BlockSpec((1, 8 * H), lambda: (0, 0)),
            pl.BlockSpec((1, 8 * H), lambda: (0, 0)),
            pl.BlockSpec((1, 8 * H), lambda: (0, 0)),
            pl.BlockSpec((8 * H, 1), lambda: (0, 0)),
            pl.BlockSpec((1, 1), lambda: (0, 0), memory_space=pltpu.SMEM),
        ],
        out_specs=pl.BlockSpec((B, 1), lambda: (0, 0)),
        out_shape=jax.ShapeDtypeStruct((B, 1), jnp.float32),
    )(s, wfc, bfc.reshape(1, 8 * H), g.reshape(1, 8 * H),
      beta.reshape(1, 8 * H), wout, bout.reshape(1, 1))


def _branch(x, ei, gid, p):
    src = ei[0]
    dst = ei[1]
    pad = E2 - E
    srcp = jnp.concatenate(
        [src, jnp.zeros((pad,), jnp.int32)]).reshape(NB2, 128)
    dstp = jnp.concatenate(
        [dst, jnp.full((pad,), N, jnp.int32)]).reshape(NB2, 128)
    zin = jnp.zeros((3128, 32), jnp.float32)
    a0 = _sc_agg(x, srcp, dstp, zin)
    z0, st0 = _t1([a0], [x], p['W_gc0'], p['b_gc0'], p['W_res0'], p['b_res0'])
    h1g = _t2a(z0, st0, p['g_bn0'], p['beta_bn0'])
    a1 = [_sc_agg(h1g[k], srcp, dstp, zin) for k in range(4)]
    z1, st1 = _t1(a1, list(h1g),
                  p['W_gc1'], p['b_gc1'], p['W_res1'], p['b_res1'])
    h2, y = _t2b(z1, st1, p['g_bn1'], p['beta_bn1'], p['W_aw'], p['b_aw'])
    ws, mx = _sc_readout(gid, y, h2)
    return ws, mx


def kernel(x1, x2, edge_index1, edge_index2, graph_ids1, graph_ids2,
           W_gc0, b_gc0, W_res0, b_res0, g_bn0, beta_bn0,
           W_gc1, b_gc1, W_res1, b_res1, g_bn1, beta_bn1,
           W_aw, b_aw, W_p1, b_p1, g_bnp, beta_bnp, W_p2, b_p2,
           W_fc, b_fc, g_bnf, beta_bnf, W_out, b_out):
    p = dict(W_gc0=W_gc0, b_gc0=b_gc0, W_res0=W_res0, b_res0=b_res0,
             g_bn0=g_bn0, beta_bn0=beta_bn0,
             W_gc1=W_gc1, b_gc1=b_gc1, W_res1=W_res1, b_res1=b_res1,
             g_bn1=g_bn1, beta_bn1=beta_bn1, W_aw=W_aw, b_aw=b_aw)
    ws1, mx1 = _branch(x1, edge_index1, graph_ids1, p)
    ws2, mx2 = _branch(x2, edge_index2, graph_ids2, p)
    ws = jnp.stack([ws1, ws2])
    mx = jnp.stack([mx1, mx2])
    s = _t3(ws, mx, W_p1, b_p1, g_bnp, beta_bnp, W_p2, b_p2)
    o = _t4(s, W_fc, b_fc, g_bnf, beta_bnf, W_out, b_out)
    return o[:, 0]


# speedup vs baseline: 3.7369x; 1.0349x over previous
import functools

import jax
import jax.numpy as jnp
from jax import lax
from jax.experimental import pallas as pl
from jax.experimental.pallas import tpu as pltpu
from jax.experimental.pallas import tpu_sc as plsc

N = 50000
E = 1600000
B = 1024
D_IN = 32
H = 128

ROWS = 1000
NB2 = 12800
E2 = NB2 * 128
NPAD = N + 128
ACC_ROWS = 50048


def _sc_agg(table, srcp, dstp, zin):
    mesh = plsc.VectorSubcoreMesh(core_axis_name="c", subcore_axis_name="s")

    @functools.partial(
        pl.kernel,
        out_type=jax.ShapeDtypeStruct((2, N, 32), jnp.float32),
        mesh=mesh,
        compiler_params=pltpu.CompilerParams(use_tc_tiling_on_sc=False),
        scratch_types=[
            pltpu.VMEM_SHARED((ACC_ROWS, 32), jnp.float32),
            pltpu.VMEM((40, 128), jnp.int32),
            pltpu.VMEM((40, 128), jnp.int32),
            pltpu.VMEM((4, 128, 32), jnp.float32),
            pltpu.SemaphoreType.DMA((4,)),
            pltpu.SemaphoreType.DMA((4,)),
        ],
    )
    def k(table_hbm, src_hbm, dst_hbm, zin_hbm, out_hbm,
          acc, idx_s, idx_d, rows, gsem, ssem):
        c = lax.axis_index("c")
        s = lax.axis_index("s")
        zoff = pl.multiple_of(s * 3128, 8)
        pltpu.sync_copy(zin_hbm, acc.at[pl.ds(zoff, 3128)])
        plsc.subcore_barrier()

        base = (c * 16 + s) * 400
        NG = 40

        def group(g, carry):
            g0 = pl.multiple_of(base + g * NG, 8)
            pltpu.sync_copy(src_hbm.at[pl.ds(g0, NG)], idx_s)
            pltpu.sync_copy(dst_hbm.at[pl.ds(g0, NG)], idx_d)
            gh = {}
            sh = {}
            for j in range(NG + 2):
                if j < NG:
                    k4 = j % 4
                    if j >= 4:
                        sh[j - 4].wait()
                    gh[j] = pltpu.async_copy(
                        table_hbm.at[idx_s.at[j]], rows.at[k4],
                        gsem.at[k4])
                if j >= 2:
                    jj = j - 2
                    gh[jj].wait()
                    sh[jj] = pltpu.async_copy(
                        rows.at[jj % 4], acc.at[idx_d.at[jj]],
                        ssem.at[jj % 4], add=True)
            for jj in range(NG - 4, NG):
                sh[jj].wait()
            return carry

        lax.fori_loop(0, 10, group, 0)
        plsc.subcore_barrier()
        ooff = pl.multiple_of(s * 3128, 8)

        @pl.when(s < 15)
        def _():
            pltpu.sync_copy(acc.at[pl.ds(ooff, 3128)],
                            out_hbm.at[c, pl.ds(ooff, 3128)])

        @pl.when(s == 15)
        def _():
            pltpu.sync_copy(acc.at[pl.ds(15 * 3128, N - 15 * 3128)],
                            out_hbm.at[c, pl.ds(15 * 3128, N - 15 * 3128)])

    return k(table, srcp, dstp, zin)


def _extract_i32(buf, i):
    win = buf[pl.ds(i, 16)]
    return win[0]


def _sc_readout(gid, y, h2):
    mesh = plsc.VectorSubcoreMesh(core_axis_name="c", subcore_axis_name="s")

    @functools.partial(
        pl.kernel,
        out_type=[
            jax.ShapeDtypeStruct((B, H), jnp.float32),
            jax.ShapeDtypeStruct((B, H), jnp.float32),
        ],
        mesh=mesh,
        scratch_types=[
            pltpu.VMEM((N + 16,), jnp.int32),
            pltpu.VMEM((128, H), jnp.float32),
            pltpu.VMEM((128, H), jnp.float32),
            pltpu.VMEM((32, H), jnp.float32),
            pltpu.VMEM((32, H), jnp.float32),
        ],
    )
    def k(gid_hbm, y_hbm, h2_hbm, ws_hbm, mx_hbm,
          gid_buf, ybuf, hbuf, ws_loc, mx_loc):
        c = lax.axis_index("c")
        s = lax.axis_index("s")
        w = c * 16 + s
        pltpu.sync_copy(gid_hbm, gid_buf.at[pl.ds(0, N)])

        def ini(r, carry):
            for cp in range(8):
                sl = pl.ds(cp * 16, 16)
                ws_loc[r, sl] = jnp.zeros((16,), jnp.float32)
                mx_loc[r, sl] = jnp.full((16,), -jnp.inf, jnp.float32)
            return carry

        lax.fori_loop(0, 32, ini, 0)

        def search(target):
            def it(t, lohi):
                lo, hi = lohi
                mid = (lo + hi) // 2
                v = _extract_i32(gid_buf, mid)
                less = v < target
                return (jnp.where(less, mid + 1, lo),
                        jnp.where(less, hi, mid))
            lo, _ = lax.fori_loop(0, 17, it, (0, N))
            return lo

        g_lo = w * 32
        r0 = search(g_lo)
        r1 = search(g_lo + 32)
        c0 = (r0 // 8) * 8
        nch = (r1 - c0 + 127) // 128

        def chunk(ci, carry):
            cstart = pl.multiple_of(c0 + ci * 128, 8)
            pltpu.sync_copy(y_hbm.at[pl.ds(cstart, 128)], ybuf)
            pltpu.sync_copy(h2_hbm.at[pl.ds(cstart, 128)], hbuf)
            rl_lo = jnp.maximum(0, r0 - cstart)
            rl_hi = jnp.minimum(128, r1 - cstart)

            def row(rl, carry2):
                g = _extract_i32(gid_buf, cstart + rl) - g_lo
                for cp in range(8):
                    sl = pl.ds(cp * 16, 16)
                    ws_loc[g, sl] = ws_loc[g, sl] + ybuf[rl, sl]
                    mx_loc[g, sl] = jnp.maximum(mx_loc[g, sl], hbuf[rl, sl])
                return carry2

            lax.fori_loop(rl_lo, rl_hi, row, 0)
            return carry

        lax.fori_loop(0, nch, chunk, 0)
        pltpu.sync_copy(ws_loc, ws_hbm.at[pl.ds(g_lo, 32)])
        pltpu.sync_copy(mx_loc, mx_hbm.at[pl.ds(g_lo, 32)])

    return k(gid, y, h2)


def _t1_kernel(*refs, ngroups, nh):
    agg_refs = refs[:ngroups]
    h_refs = refs[ngroups:ngroups + nh]
    wg_ref, bg_ref, wr_ref, br_ref, z_ref, st_ref = refs[ngroups + nh:]
    i = pl.program_id(0)
    agg = jnp.concatenate([r[0] + r[1] for r in agg_refs], axis=1)
    h = jnp.concatenate([r[...] for r in h_refs], axis=1)
    a = jnp.dot(agg, wg_ref[...], preferred_element_type=jnp.float32)
    a = jnp.maximum(a + bg_ref[...], 0.0)
    r = jnp.dot(h, wr_ref[...], preferred_element_type=jnp.float32)
    r = jnp.maximum(r + br_ref[...], 0.0)
    z = a + r
    z_ref[...] = z

    @pl.when(i == 0)
    def _():
        st_ref[...] = jnp.zeros_like(st_ref)

    st_ref[0:1, :] += jnp.sum(z, axis=0, keepdims=True)
    st_ref[1:2, :] += jnp.sum(z * z, axis=0, keepdims=True)


def _t1(agg_parts, hs, wg, bg, wr, br):
    ngroups = len(agg_parts)
    nh = len(hs)
    din = 32 * nh
    grid = (N // ROWS,)
    in_specs = (
        [pl.BlockSpec((2, ROWS, 32), lambda i: (0, i, 0))] * ngroups
        + [pl.BlockSpec((ROWS, 32), lambda i: (i, 0))] * nh
        + [
            pl.BlockSpec((din, H), lambda i: (0, 0)),
            pl.BlockSpec((1, H), lambda i: (0, 0)),
            pl.BlockSpec((din, H), lambda i: (0, 0)),
            pl.BlockSpec((1, H), lambda i: (0, 0)),
        ]
    )
    out_specs = [
        pl.BlockSpec((ROWS, H), lambda i: (i, 0)),
        pl.BlockSpec((8, H), lambda i: (0, 0)),
    ]
    z, st = pl.pallas_call(
        functools.partial(_t1_kernel, ngroups=ngroups, nh=nh),
        grid=grid,
        in_specs=in_specs,
        out_specs=out_specs,
        out_shape=[
            jax.ShapeDtypeStruct((N, H), jnp.float32),
            jax.ShapeDtypeStruct((8, H), jnp.float32),
        ],
    )(*agg_parts, *hs, wg, bg.reshape(1, H), wr, br.reshape(1, H))
    return z, st


def _t2a_kernel(z_ref, st_ref, g_ref, beta_ref, *hg_refs):
    s = st_ref[0:1, :]
    ss = st_ref[1:2, :]
    m = s / N
    v = ss / N - m * m
    scale = g_ref[...] * lax.rsqrt(v + 1e-5)
    shift = beta_ref[...] - m * scale
    hcur = z_ref[...] * scale + shift
    for k in range(4):
        hg_refs[k][...] = hcur[:, 32 * k:32 * (k + 1)]


def _t2a(z, st, g, beta):
    return pl.pallas_call(
        _t2a_kernel,
        grid=(N // ROWS,),
        in_specs=[
            pl.BlockSpec((ROWS, H), lambda i: (i, 0)),
            pl.BlockSpec((8, H), lambda i: (0, 0)),
            pl.BlockSpec((1, H), lambda i: (0, 0)),
            pl.BlockSpec((1, H), lambda i: (0, 0)),
        ],
        out_specs=[pl.BlockSpec((ROWS, 32), lambda i: (i, 0))] * 4,
        out_shape=[jax.ShapeDtypeStruct((N, 32), jnp.float32)] * 4,
    )(z, st, g.reshape(1, H), beta.reshape(1, H))


def _t2b_kernel(z_ref, st_ref, g_ref, beta_ref, waw_ref, baw_ref,
                h2_ref, y_ref):
    s = st_ref[0:1, :]
    ss = st_ref[1:2, :]
    m = s / N
    v = ss / N - m * m
    scale = g_ref[...] * lax.rsqrt(v + 1e-5)
    shift = beta_ref[...] - m * scale
    h2 = z_ref[...] * scale + shift
    h2_ref[...] = h2
    wv = jnp.dot(h2, waw_ref[...], preferred_element_type=jnp.float32)
    wv = jax.nn.sigmoid(wv + baw_ref[0, 0])
    y_ref[...] = wv * h2


def _t2b(z, st, g, beta, waw, baw):
    return pl.pallas_call(
        _t2b_kernel,
        grid=(N // ROWS,),
        in_specs=[
            pl.BlockSpec((ROWS, H), lambda i: (i, 0)),
            pl.BlockSpec((8, H), lambda i: (0, 0)),
            pl.BlockSpec((1, H), lambda i: (0, 0)),
            pl.BlockSpec((1, H), lambda i: (0, 0)),
            pl.BlockSpec((H, 1), lambda i: (0, 0)),
            pl.BlockSpec((1, 1), lambda i: (0, 0), memory_space=pltpu.SMEM),
        ],
        out_specs=[
            pl.BlockSpec((ROWS, H), lambda i: (i, 0)),
            pl.BlockSpec((ROWS, H), lambda i: (i, 0)),
        ],
        out_shape=[
            jax.ShapeDtypeStruct((NPAD, H), jnp.float32),
            jax.ShapeDtypeStruct((NPAD, H), jnp.float32),
        ],
    )(z, st, g.reshape(1, H), beta.reshape(1, H), waw, baw.reshape(1, 1))


def _t3_kernel(ws_ref, mx_ref, wp1_ref, bp1_ref, g_ref, beta_ref,
               wp2_ref, bp2_ref, s_ref):
    ws = ws_ref[0]
    mx = mx_ref[0]
    mx = jnp.where(jnp.isfinite(mx), mx, 0.0)
    gf = jnp.concatenate([ws, mx], axis=1)
    q = jnp.dot(gf, wp1_ref[...], preferred_element_type=jnp.float32)
    q = jnp.maximum(q + bp1_ref[...], 0.0)
    m = jnp.mean(q, axis=0, keepdims=True)
    v = jnp.mean(q * q, axis=0, keepdims=True) - m * m
    q = g_ref[...] * (q - m) * lax.rsqrt(v + 1e-5) + beta_ref[...]
    s_ref[0] = jnp.dot(q, wp2_ref[...],
                       preferred_element_type=jnp.float32) + bp2_ref[...]


def _t3(ws, mx, wp1, bp1, g, beta, wp2, bp2):
    h2c = 2 * H
    return pl.pallas_call(
        _t3_kernel,
        grid=(2,),
        in_specs=[
            pl.BlockSpec((1, B, H), lambda i: (i, 0, 0)),
            pl.BlockSpec((1, B, H), lambda i: (i, 0, 0)),
            pl.BlockSpec((h2c, h2c), lambda i: (0, 0)),
            pl.BlockSpec((1, h2c), lambda i: (0, 0)),
            pl.BlockSpec((1, h2c), lambda i: (0, 0)),
            pl.BlockSpec((1, h2c), lambda i: (0, 0)),
            pl.BlockSpec((h2c, h2c), lambda i: (0, 0)),
            pl.BlockSpec((1, h2c), lambda i: (0, 0)),
        ],
        out_specs=pl.BlockSpec((1, B, h2c), lambda i: (i, 0, 0)),
        out_shape=jax.ShapeDtypeStruct((2, B, h2c), jnp.float32),
    )(ws, mx, wp1, bp1.reshape(1, h2c), g.reshape(1, h2c),
      beta.reshape(1, h2c), wp2, bp2.reshape(1, h2c))


def _t4_kernel(s_ref, wfc_ref, bfc_ref, g_ref, beta_ref, wout_ref, bout_ref,
               o_ref):
    gf = jnp.concatenate([s_ref[0], s_ref[1]], axis=1)
    z = jnp.dot(gf, wfc_ref[...], preferred_element_type=jnp.float32)
    z = z + bfc_ref[...]
    m = jnp.mean(z, axis=0, keepdims=True)
    v = jnp.mean(z * z, axis=0, keepdims=True) - m * m
    z = g_ref[...] * (z - m) * lax.rsqrt(v + 1e-5) + beta_ref[...]
    z = jnp.maximum(z, 0.0)
    o = jnp.dot(z, wout_ref[...], preferred_element_type=jnp.float32)
    o_ref[...] = jax.nn.sigmoid(o + bout_ref[0, 0])


def _t4(s, wfc, bfc, g, beta, wout, bout):
    return pl.pallas_call(
        _t4_kernel,
        in_specs=[
            pl.BlockSpec((2, B, 2 * H), lambda: (0, 0, 0)),
            pl.BlockSpec((4 * H, 8 * H), lambda: (0, 0)),
            pl.BlockSpec((1, 8 * H), lambda: (0, 0)),
            pl.BlockSpec((1, 8 * H), lambda: (0, 0)),
            pl.BlockSpec((1, 8 * H), lambda: (0, 0)),
            pl.BlockSpec((8 * H, 1), lambda: (0, 0)),
            pl.BlockSpec((1, 1), lambda: (0, 0), memory_space=pltpu.SMEM),
        ],
        out_specs=pl.BlockSpec((B, 1), lambda: (0, 0)),
        out_shape=jax.ShapeDtypeStruct((B, 1), jnp.float32),
    )(s, wfc, bfc.reshape(1, 8 * H), g.reshape(1, 8 * H),
      beta.reshape(1, 8 * H), wout, bout.reshape(1, 1))


def _branch(x, ei, gid, p):
    src = ei[0]
    dst = ei[1]
    pad = E2 - E
    srcp = jnp.concatenate(
        [src, jnp.zeros((pad,), jnp.int32)]).reshape(NB2, 128)
    dstp = jnp.concatenate(
        [dst, jnp.full((pad,), N, jnp.int32)]).reshape(NB2, 128)
    zin = jnp.zeros((3128, 32), jnp.float32)
    a0 = _sc_agg(x, srcp, dstp, zin)
    z0, st0 = _t1([a0], [x], p['W_gc0'], p['b_gc0'], p['W_res0'], p['b_res0'])
    h1g = _t2a(z0, st0, p['g_bn0'], p['beta_bn0'])
    a1 = [_sc_agg(h1g[k], srcp, dstp, zin) for k in range(4)]
    z1, st1 = _t1(a1, list(h1g),
                  p['W_gc1'], p['b_gc1'], p['W_res1'], p['b_res1'])
    h2, y = _t2b(z1, st1, p['g_bn1'], p['beta_bn1'], p['W_aw'], p['b_aw'])
    ws, mx = _sc_readout(gid, y, h2)
    return ws, mx


def kernel(x1, x2, edge_index1, edge_index2, graph_ids1, graph_ids2,
           W_gc0, b_gc0, W_res0, b_res0, g_bn0, beta_bn0,
           W_gc1, b_gc1, W_res1, b_res1, g_bn1, beta_bn1,
           W_aw, b_aw, W_p1, b_p1, g_bnp, beta_bnp, W_p2, b_p2,
           W_fc, b_fc, g_bnf, beta_bnf, W_out, b_out):
    p = dict(W_gc0=W_gc0, b_gc0=b_gc0, W_res0=W_res0, b_res0=b_res0,
             g_bn0=g_bn0, beta_bn0=beta_bn0,
             W_gc1=W_gc1, b_gc1=b_gc1, W_res1=W_res1, b_res1=b_res1,
             g_bn1=g_bn1, beta_bn1=beta_bn1, W_aw=W_aw, b_aw=b_aw)
    ws1, mx1 = _branch(x1, edge_index1, graph_ids1, p)
    ws2, mx2 = _branch(x2, edge_index2, graph_ids2, p)
    ws = jnp.stack([ws1, ws2])
    mx = jnp.stack([mx1, mx2])
    s = _t3(ws, mx, W_p1, b_p1, g_bnp, beta_bnp, W_p2, b_p2)
    o = _t4(s, W_fc, b_fc, g_bnf, beta_bnf, W_out, b_out)
    return o[:, 0]
